# baseline (speedup 1.0000x reference)
import functools
import numpy as np
import jax
import jax.numpy as jnp
from jax import lax
from jax.experimental import pallas as pl
from jax.experimental.pallas import tpu as pltpu
from jax._src.pallas.mosaic import sc_core as plsc


def _np_threefry2x32(k0, k1, x0, x1):
    def rotl(x, d):
        return ((x << np.uint32(d)) | (x >> np.uint32(32 - d))).astype(np.uint32)
    ks = [np.uint32(k0), np.uint32(k1),
          np.uint32(np.uint32(k0) ^ np.uint32(k1) ^ np.uint32(0x1BD11BDA))]
    x0 = (x0 + ks[0]).astype(np.uint32)
    x1 = (x1 + ks[1]).astype(np.uint32)
    rots = [[13, 15, 26, 6], [17, 29, 16, 24]]
    for i in range(5):
        for r in rots[i % 2]:
            x0 = (x0 + x1).astype(np.uint32)
            x1 = rotl(x1, r)
            x1 = (x0 ^ x1).astype(np.uint32)
        x0 = (x0 + ks[(i + 1) % 3]).astype(np.uint32)
        x1 = (x1 + ks[(i + 2) % 3] + np.uint32(i + 1)).astype(np.uint32)
    return x0, x1


def _child_key(seed_hi, seed_lo, i):
    a, b = _np_threefry2x32(seed_hi, seed_lo,
                            np.array([0], np.uint32), np.array([i], np.uint32))
    return int(a[0]), int(b[0])


_K1 = _child_key(0, 42, 0)
_K2 = _child_key(0, 42, 1)

_TINY = float(np.finfo(np.float32).tiny)
_U2_MIN = np.float32(1e-5)
_U2_SPAN = np.float32(np.float32(1.0 - 1e-5) - np.float32(1e-5))

_ROT = ((13, 15, 26, 6), (17, 29, 16, 24))


def _wrap_i32(v):
    v &= 0xFFFFFFFF
    return v - (1 << 32) if v >= (1 << 31) else v


def _rotl(x, r):
    return lax.shift_left(x, jnp.int32(r)) | lax.shift_right_logical(x, jnp.int32(32 - r))


def _key_sched(kpair):
    k0, k1 = kpair
    ks = (_wrap_i32(k0), _wrap_i32(k1), _wrap_i32(k0 ^ k1 ^ 0x1BD11BDA))
    inj = [(ks[(i + 1) % 3], _wrap_i32(ks[(i + 2) % 3] + i + 1)) for i in range(5)]
    return ks[0], ks[1], inj


_SCHED1 = _key_sched(_K1)
_SCHED2 = _key_sched(_K2)


def _threefry_bits(sched, x1):
    ks0, _ks1, inj = sched
    x0 = jnp.full_like(x1, jnp.int32(ks0))
    for i in range(5):
        for r in _ROT[i % 2]:
            x0 = x0 + x1
            x1 = _rotl(x1, r)
            x1 = x0 ^ x1
        x0 = x0 + jnp.int32(inj[i][0])
        x1 = x1 + jnp.int32(inj[i][1])
    return x0 ^ x1


def _bits_to_unit(bits):
    fb = lax.shift_right_logical(bits, jnp.int32(9)) | jnp.int32(0x3F800000)
    return lax.bitcast_convert_type(fb, jnp.float32) - jnp.float32(1.0)


def _body(l_ref, cnt1_ref, cnt2_ref, out_ref, *, n, T, BW, TW):
    i = pl.program_id(0)
    j = pl.program_id(1)
    lb = l_ref[...]

    base1 = (i * BW * n) * T + j * TW
    x1 = cnt1_ref[...] + base1
    bits = _threefry_bits(_SCHED1, x1)
    ug = jnp.maximum(_bits_to_unit(bits), jnp.float32(_TINY))
    gum = -jnp.log(-jnp.log(ug))

    bv = lb[0] + gum[0]
    bm = lb[n]
    bl = lb[2 * n]
    for g in range(1, n):
        vg = lb[g] + gum[g]
        upd = vg > bv
        bv = jnp.where(upd, vg, bv)
        bm = jnp.where(upd, lb[n + g], bm)
        bl = jnp.where(upd, lb[2 * n + g], bl)

    sel_mean = jnp.tanh(bm)
    sel_lv = jnp.float32(-7.0) * jax.nn.sigmoid(bl)

    bits2 = cnt2_ref[...]
    u2 = jnp.maximum(_U2_MIN, _bits_to_unit(bits2) * _U2_SPAN + _U2_MIN)

    noise = jnp.exp(sel_lv) * (jnp.log(u2) - jnp.log(jnp.float32(1.0) - u2))
    x = jnp.clip(sel_mean + noise, -1.0, 1.0)
    y = jnp.round((x + jnp.float32(1.0)) * jnp.float32(127.5))
    out_ref[...] = y.astype(jnp.int32)


def _counter_consts(n, T, BW, TW):
    g = np.arange(n, dtype=np.int64)[:, None, None]
    b = np.arange(BW, dtype=np.int64)[None, :, None]
    c = np.arange(TW, dtype=np.int64)[None, None, :]
    cnt1 = (b * n + g) * T + c + _SCHED1[1]
    wrap = lambda a: ((a & 0xFFFFFFFF) ^ (1 << 31)) - (1 << 31)
    return jnp.asarray(wrap(cnt1), jnp.int32)


def _sc_u2_body(out_ref, scratch_ref, *, B, T):
    c_id = lax.axis_index("c")
    s_id = lax.axis_index("s")
    sub = c_id * 16 + s_id
    tb = sub // 8
    tcg = (sub % 8) * 8
    iota = lax.iota(jnp.int32, 16)
    ks1 = jnp.int32(_SCHED2[1])
    for s8 in range(8):
        b = tb * 8 + s8

        def chunk(q, carry):
            kt = q // 8
            c16 = q % 8
            t0 = (tcg + kt) * 128 + c16 * 16
            x1 = (b * T + t0) + ks1 + iota
            bits = _threefry_bits(_SCHED2, x1)
            scratch_ref[s8, pl.ds(kt * 128 + c16 * 16, 16)] = bits
            return carry

        jax.lax.fori_loop(0, 64, chunk, 0)
    pltpu.sync_copy(scratch_ref,
                    out_ref.at[pl.ds(tb * 8, 8), pl.ds(tcg * 128, 1024)])


def kernel(l):
    B, C, T = l.shape
    n = C // 3
    BW, TW = 8, 1024
    lt = jnp.transpose(l, (1, 0, 2))
    cnt1 = _counter_consts(n, T, BW, TW)
    u2bits = pl.kernel(
        functools.partial(_sc_u2_body, B=B, T=T),
        out_type=jax.ShapeDtypeStruct((B, T), jnp.int32),
        mesh=plsc.VectorSubcoreMesh(core_axis_name="c", subcore_axis_name="s"),
        scratch_types=[pltpu.VMEM((8, 1024), jnp.int32)],
    )()
    out = pl.pallas_call(
        functools.partial(_body, n=n, T=T, BW=BW, TW=TW),
        grid=(B // BW, T // TW),
        in_specs=[
            pl.BlockSpec((C, BW, TW), lambda i, j: (0, i, j)),
            pl.BlockSpec((n, BW, TW), lambda i, j: (0, 0, 0)),
            pl.BlockSpec((BW, TW), lambda i, j: (i, j)),
        ],
        out_specs=pl.BlockSpec((BW, TW), lambda i, j: (i, j)),
        out_shape=jax.ShapeDtypeStruct((B, T), jnp.int32),
    )(lt, cnt1, u2bits)
    return out


# speedup vs baseline: 1.1891x; 1.1891x over previous
import functools
import numpy as np
import jax
import jax.numpy as jnp
from jax import lax
from jax.experimental import pallas as pl


def _np_threefry2x32(k0, k1, x0, x1):
    def rotl(x, d):
        return ((x << np.uint32(d)) | (x >> np.uint32(32 - d))).astype(np.uint32)
    ks = [np.uint32(k0), np.uint32(k1),
          np.uint32(np.uint32(k0) ^ np.uint32(k1) ^ np.uint32(0x1BD11BDA))]
    x0 = (x0 + ks[0]).astype(np.uint32)
    x1 = (x1 + ks[1]).astype(np.uint32)
    rots = [[13, 15, 26, 6], [17, 29, 16, 24]]
    for i in range(5):
        for r in rots[i % 2]:
            x0 = (x0 + x1).astype(np.uint32)
            x1 = rotl(x1, r)
            x1 = (x0 ^ x1).astype(np.uint32)
        x0 = (x0 + ks[(i + 1) % 3]).astype(np.uint32)
        x1 = (x1 + ks[(i + 2) % 3] + np.uint32(i + 1)).astype(np.uint32)
    return x0, x1


def _child_key(seed_hi, seed_lo, i):
    a, b = _np_threefry2x32(seed_hi, seed_lo,
                            np.array([0], np.uint32), np.array([i], np.uint32))
    return int(a[0]), int(b[0])


_K1 = _child_key(0, 42, 0)
_K2 = _child_key(0, 42, 1)

_TINY = float(np.finfo(np.float32).tiny)
_U2_MIN = np.float32(1e-5)
_U2_SPAN = np.float32(np.float32(1.0 - 1e-5) - np.float32(1e-5))

_ROT = ((13, 15, 26, 6), (17, 29, 16, 24))


def _wrap_i32(v):
    v &= 0xFFFFFFFF
    return v - (1 << 32) if v >= (1 << 31) else v


def _rotl(x, r):
    return lax.shift_left(x, jnp.int32(r)) | lax.shift_right_logical(x, jnp.int32(32 - r))


def _key_sched(kpair):
    k0, k1 = kpair
    ks = (_wrap_i32(k0), _wrap_i32(k1), _wrap_i32(k0 ^ k1 ^ 0x1BD11BDA))
    inj = [(ks[(i + 1) % 3], _wrap_i32(ks[(i + 2) % 3] + i + 1)) for i in range(5)]
    return ks[0], ks[1], inj


_SCHED1 = _key_sched(_K1)
_SCHED2 = _key_sched(_K2)


def _threefry_bits(sched, x1):
    ks0, _ks1, inj = sched
    x0 = jnp.full_like(x1, jnp.int32(ks0))
    for i in range(5):
        for r in _ROT[i % 2]:
            x0 = x0 + x1
            x1 = _rotl(x1, r)
            x1 = x0 ^ x1
        x0 = x0 + jnp.int32(inj[i][0])
        x1 = x1 + jnp.int32(inj[i][1])
    return x0 ^ x1


def _bits_to_unit(bits):
    fb = lax.shift_right_logical(bits, jnp.int32(9)) | jnp.int32(0x3F800000)
    return lax.bitcast_convert_type(fb, jnp.float32) - jnp.float32(1.0)


def _body(l_ref, cnt1_ref, cnt2_ref, out_ref, *, n, T, BW, TW):
    i = pl.program_id(0)
    j = pl.program_id(1)
    lb = l_ref[...]

    base1 = (i * BW * n) * T + j * TW
    x1 = cnt1_ref[...] + base1
    bits = _threefry_bits(_SCHED1, x1)
    ug = jnp.maximum(_bits_to_unit(bits), jnp.float32(_TINY))
    gum = -jnp.log(-jnp.log(ug))

    bv = lb[0] + gum[0]
    bm = lb[n]
    bl = lb[2 * n]
    for g in range(1, n):
        vg = lb[g] + gum[g]
        upd = vg > bv
        bv = jnp.where(upd, vg, bv)
        bm = jnp.where(upd, lb[n + g], bm)
        bl = jnp.where(upd, lb[2 * n + g], bl)

    sel_mean = jnp.tanh(bm)
    sel_lv = jnp.float32(-7.0) * jax.nn.sigmoid(bl)

    base2 = (i * BW) * T + j * TW
    bits2 = _threefry_bits(_SCHED2, cnt2_ref[...] + base2)
    u2 = jnp.maximum(_U2_MIN, _bits_to_unit(bits2) * _U2_SPAN + _U2_MIN)

    noise = jnp.exp(sel_lv) * (jnp.log(u2) - jnp.log(jnp.float32(1.0) - u2))
    x = jnp.clip(sel_mean + noise, -1.0, 1.0)
    y = jnp.round((x + jnp.float32(1.0)) * jnp.float32(127.5))
    out_ref[...] = y.astype(jnp.int32)


def _counter_consts(n, T, BW, TW):
    g = np.arange(n, dtype=np.int64)[:, None, None]
    b = np.arange(BW, dtype=np.int64)[None, :, None]
    c = np.arange(TW, dtype=np.int64)[None, None, :]
    cnt1 = (b * n + g) * T + c + _SCHED1[1]
    cnt2 = (np.arange(BW, dtype=np.int64)[:, None] * T
            + np.arange(TW, dtype=np.int64)[None, :] + _SCHED2[1])
    wrap = lambda a: ((a & 0xFFFFFFFF) ^ (1 << 31)) - (1 << 31)
    return (jnp.asarray(wrap(cnt1), jnp.int32), jnp.asarray(wrap(cnt2), jnp.int32))


def kernel(l):
    B, C, T = l.shape
    n = C // 3
    BW, TW = 8, 1024
    lt = jnp.transpose(l, (1, 0, 2))
    cnt1, cnt2 = _counter_consts(n, T, BW, TW)
    out = pl.pallas_call(
        functools.partial(_body, n=n, T=T, BW=BW, TW=TW),
        grid=(B // BW, T // TW),
        in_specs=[
            pl.BlockSpec((C, BW, TW), lambda i, j: (0, i, j)),
            pl.BlockSpec((n, BW, TW), lambda i, j: (0, 0, 0)),
            pl.BlockSpec((BW, TW), lambda i, j: (0, 0)),
        ],
        out_specs=pl.BlockSpec((BW, TW), lambda i, j: (i, j)),
        out_shape=jax.ShapeDtypeStruct((B, T), jnp.int32),
    )(lt, cnt1, cnt2)
    return out
